# Initial kernel scaffold; baseline (speedup 1.0000x reference)
import jax
import jax.numpy as jnp
from jax.experimental import pallas as pl


def kernel(node_features, grouped_messages, W, b):
    raise NotImplementedError("write your pallas kernel here")


# speedup vs baseline: 2.1136x; 2.1136x over previous
import jax
import jax.numpy as jnp
from jax.experimental import pallas as pl

_TILE = 1000


def _agg_kernel(nf_ref, msg_ref, w_ref, b_ref, out_ref):
    w = w_ref[...]
    nf = nf_ref[...]
    bias = b_ref[...]
    g_groups = msg_ref.shape[0]
    for g in range(g_groups):
        proj = jnp.dot(msg_ref[g], w, preferred_element_type=jnp.float32)
        out_ref[g] = jnp.maximum(0.5 * (proj + bias + nf), 0.0)


def kernel(node_features, grouped_messages, W, b):
    G, N, d_msg = grouped_messages.shape
    d_node = node_features.shape[1]
    b2 = b.reshape(1, d_node)

    out = pl.pallas_call(
        _agg_kernel,
        grid=(N // _TILE,),
        in_specs=[
            pl.BlockSpec((_TILE, d_node), lambda i: (i, 0)),
            pl.BlockSpec((G, _TILE, d_msg), lambda i: (0, i, 0)),
            pl.BlockSpec((d_msg, d_node), lambda i: (0, 0)),
            pl.BlockSpec((1, d_node), lambda i: (0, 0)),
        ],
        out_specs=pl.BlockSpec((G, _TILE, d_node), lambda i: (0, i, 0)),
        out_shape=jax.ShapeDtypeStruct((G, N, d_node), jnp.float32),
    )(node_features, grouped_messages, W, b2)
    return out.reshape(G * N, d_node)


# speedup vs baseline: 2.5340x; 1.1989x over previous
import jax
import jax.numpy as jnp
from jax.experimental import pallas as pl

_TILE = 2000


def _agg_kernel(nf_ref, msg_ref, w_ref, b_ref, out_ref):
    w = w_ref[...]
    nf = nf_ref[...]
    bias = b_ref[...]
    g_groups = msg_ref.shape[0]
    for g in range(g_groups):
        proj = jnp.dot(msg_ref[g], w, preferred_element_type=jnp.float32)
        out_ref[g] = jnp.maximum(0.5 * (proj + bias + nf), 0.0)


def kernel(node_features, grouped_messages, W, b):
    G, N, d_msg = grouped_messages.shape
    d_node = node_features.shape[1]
    b2 = b.reshape(1, d_node)

    out = pl.pallas_call(
        _agg_kernel,
        grid=(N // _TILE,),
        in_specs=[
            pl.BlockSpec((_TILE, d_node), lambda i: (i, 0)),
            pl.BlockSpec((G, _TILE, d_msg), lambda i: (0, i, 0)),
            pl.BlockSpec((d_msg, d_node), lambda i: (0, 0)),
            pl.BlockSpec((1, d_node), lambda i: (0, 0)),
        ],
        out_specs=pl.BlockSpec((G, _TILE, d_node), lambda i: (0, i, 0)),
        out_shape=jax.ShapeDtypeStruct((G, N, d_node), jnp.float32),
    )(node_features, grouped_messages, W, b2)
    return out.reshape(G * N, d_node)


# speedup vs baseline: 2.5745x; 1.0160x over previous
import jax
import jax.numpy as jnp
from jax.experimental import pallas as pl

_TILE = 4000


def _agg_kernel(nf_ref, msg_ref, w_ref, b_ref, out_ref):
    w = w_ref[...]
    nf = nf_ref[...]
    bias = b_ref[...]
    g_groups = msg_ref.shape[0]
    for g in range(g_groups):
        proj = jnp.dot(msg_ref[g], w, preferred_element_type=jnp.float32)
        out_ref[g] = jnp.maximum(0.5 * (proj + bias + nf), 0.0)


def kernel(node_features, grouped_messages, W, b):
    G, N, d_msg = grouped_messages.shape
    d_node = node_features.shape[1]
    b2 = b.reshape(1, d_node)

    out = pl.pallas_call(
        _agg_kernel,
        grid=(N // _TILE,),
        in_specs=[
            pl.BlockSpec((_TILE, d_node), lambda i: (i, 0)),
            pl.BlockSpec((G, _TILE, d_msg), lambda i: (0, i, 0)),
            pl.BlockSpec((d_msg, d_node), lambda i: (0, 0)),
            pl.BlockSpec((1, d_node), lambda i: (0, 0)),
        ],
        out_specs=pl.BlockSpec((G, _TILE, d_node), lambda i: (0, i, 0)),
        out_shape=jax.ShapeDtypeStruct((G, N, d_node), jnp.float32),
    )(node_features, grouped_messages, W, b2)
    return out.reshape(G * N, d_node)


# speedup vs baseline: 2.5766x; 1.0008x over previous
import jax
import jax.numpy as jnp
from jax.experimental import pallas as pl
from jax.experimental.pallas import tpu as pltpu

_TILE = 4000


def _agg_kernel(nf_ref, msg_ref, w_ref, b_ref, out_ref):
    w = w_ref[...]
    nf = nf_ref[...]
    bias = b_ref[...]
    g_groups = msg_ref.shape[0]
    for g in range(g_groups):
        proj = jnp.dot(msg_ref[g], w, preferred_element_type=jnp.float32)
        out_ref[g] = jnp.maximum(0.5 * (proj + bias + nf), 0.0)


def kernel(node_features, grouped_messages, W, b):
    G, N, d_msg = grouped_messages.shape
    d_node = node_features.shape[1]
    b2 = b.reshape(1, d_node)

    out = pl.pallas_call(
        _agg_kernel,
        grid=(N // _TILE,),
        in_specs=[
            pl.BlockSpec((_TILE, d_node), lambda i: (i, 0)),
            pl.BlockSpec((G, _TILE, d_msg), lambda i: (0, i, 0)),
            pl.BlockSpec((d_msg, d_node), lambda i: (0, 0)),
            pl.BlockSpec((1, d_node), lambda i: (0, 0)),
        ],
        out_specs=pl.BlockSpec((G, _TILE, d_node), lambda i: (0, i, 0)),
        out_shape=jax.ShapeDtypeStruct((G, N, d_node), jnp.float32),
        compiler_params=pltpu.CompilerParams(
            dimension_semantics=("parallel",),
        ),
    )(node_features, grouped_messages, W, b2)
    return out.reshape(G * N, d_node)
